# Initial kernel scaffold; baseline (speedup 1.0000x reference)
import jax
import jax.numpy as jnp
from jax.experimental import pallas as pl


def kernel(x, edge_index, batch, edge_attr, Wn, bn_, We, be, eps, W1, b1, W2, b2, gamma, beta, Wc1, bc1, Wc2, bc2):
    raise NotImplementedError("write your pallas kernel here")


# speedup vs baseline: 3.5837x; 3.5837x over previous
import functools

import jax
import jax.numpy as jnp
from jax import lax
from jax.experimental import pallas as pl
from jax.experimental.pallas import tpu as pltpu
from jax.experimental.pallas import tpu_sc as plsc

N = 10000
E = 320000
D = 128
DE = 16
H = 128
L = 5
G = 128
C = 10

NC = 2
NS = 16
NW = NC * NS
CH = 64
EPT = E // NW
NCHUNK = 2 * (-(-EPT // (2 * CH)))
EPT_PAD = NCHUNK * CH
E_PAD = NW * EPT_PAD
AGG_ROWS = 10240
ZROWS = AGG_ROWS // NS

_F32 = jnp.float32


@functools.lru_cache(maxsize=None)
def _make_message_pass():
    mesh = plsc.VectorSubcoreMesh(
        core_axis_name="c", subcore_axis_name="s",
        num_cores=NC, num_subcores=NS)

    @functools.partial(
        pl.kernel,
        out_type=jax.ShapeDtypeStruct((NC, AGG_ROWS, H), _F32),
        mesh=mesh,
        scratch_types=[
            pltpu.VMEM_SHARED((AGG_ROWS, H), _F32),
            pltpu.VMEM((CH, H), _F32),
            pltpu.VMEM((CH, H), _F32),
            pltpu.VMEM((CH, H), _F32),
            pltpu.VMEM((CH, H), _F32),
            pltpu.VMEM((CH,), jnp.int32),
            pltpu.VMEM((CH,), jnp.int32),
            pltpu.VMEM((CH,), jnp.int32),
            pltpu.VMEM((CH,), jnp.int32),
        ] + [pltpu.SemaphoreType.DMA] * 8,
    )
    def message_pass(h_hbm, ea_hbm, src_hbm, dst_hbm, z_hbm, out_hbm,
                     agg, hb0, hb1, eb0, eb1, sv0, sv1, dv0, dv1,
                     gs0, gs1, es0, es1, ss0, ss1, ds0, ds1):
        c = lax.axis_index("c")
        s = lax.axis_index("s")
        wid = c * NS + s

        pltpu.sync_copy(z_hbm, hb0)
        for k in range(ZROWS // CH):
            pltpu.sync_copy(hb0, agg.at[pl.ds(s * ZROWS + k * CH, CH)])
        plsc.subcore_barrier()

        base = wid * EPT_PAD
        hbufs = (hb0, hb1)
        ebufs = (eb0, eb1)
        srcv = (sv0, sv1)
        dstv = (dv0, dv1)
        gsems = (gs0, gs1)
        esems = (es0, es1)
        ssems = (ss0, ss1)
        dsems = (ds0, ds1)

        def sidx_copy(j, sl):
            return pltpu.make_async_copy(
                src_hbm.at[pl.ds(base + j * CH, CH)], srcv[sl], ssems[sl])

        def didx_copy(j, sl):
            return pltpu.make_async_copy(
                dst_hbm.at[pl.ds(base + j * CH, CH)], dstv[sl], dsems[sl])

        def gather_copy(sl):
            return pltpu.make_async_copy(
                h_hbm.at[srcv[sl]], hbufs[sl], gsems[sl])

        def ea_copy(j, sl):
            return pltpu.make_async_copy(
                ea_hbm.at[pl.ds(base + j * CH, CH)], ebufs[sl], esems[sl])

        def compute(sl):
            hb = hbufs[sl]
            eb = ebufs[sl]

            def row(r, carry):
                for q in range(H // 16):
                    v = hb[r, pl.ds(q * 16, 16)] + eb[r, pl.ds(q * 16, 16)]
                    hb[r, pl.ds(q * 16, 16)] = jnp.maximum(v, 0.0)
                return carry

            lax.fori_loop(0, CH, row, 0)

        def step(j, sl, has_next, has_next2):
            nsl = 1 - sl
            if has_next:
                sidx_copy(j + 1, nsl).wait()
                didx_copy(j + 1, nsl).wait()
                gather_copy(nsl).start()
                ea_copy(j + 1, nsl).start()
            gather_copy(sl).wait()
            ea_copy(j, sl).wait()
            if has_next2:
                sidx_copy(j + 2, sl).start()
            compute(sl)
            pltpu.sync_copy(hbufs[sl], agg.at[dstv[sl]], add=True)
            if has_next2:
                didx_copy(j + 2, sl).start()

        def pair(i, carry):
            j0 = 2 * i
            step(j0, 0, True, True)
            step(j0 + 1, 1, True, True)
            return carry

        sidx_copy(0, 0).start()
        didx_copy(0, 0).start()
        sidx_copy(1, 1).start()
        didx_copy(1, 1).start()
        sidx_copy(0, 0).wait()
        didx_copy(0, 0).wait()
        gather_copy(0).start()
        ea_copy(0, 0).start()
        lax.fori_loop(0, NCHUNK // 2 - 1, pair, 0)
        step(NCHUNK - 2, 0, True, False)
        step(NCHUNK - 1, 1, False, False)

        plsc.subcore_barrier()
        for k in range(ZROWS // CH):
            off = s * ZROWS + k * CH
            pltpu.sync_copy(agg.at[pl.ds(off, CH)], hb0)
            pltpu.sync_copy(hb0, out_hbm.at[c, pl.ds(off, CH)])

    return message_pass


_RB = 1000
_NBLK = N // _RB
_EB = 2048


def _proj_node(x_ref, w_ref, b_ref, o_ref):
    o_ref[...] = (
        jnp.dot(x_ref[...], w_ref[...], preferred_element_type=_F32)
        + b_ref[...])


def _proj_edge(a_ref, w_ref, b_ref, o_ref):
    o_ref[...] = (
        jnp.dot(a_ref[...], w_ref[...], preferred_element_type=_F32)
        + b_ref[...])


def _mlp_stats(scl_ref, h_ref, a0_ref, a1_ref, w1_ref, b1_ref, w2_ref, b2_ref,
               h2_ref, st_ref):
    z = scl_ref[0, 0] * h_ref[...] + a0_ref[0] + a1_ref[0]
    t = jnp.maximum(
        jnp.dot(z, w1_ref[...], preferred_element_type=_F32) + b1_ref[...], 0.0)
    h2 = jnp.dot(t, w2_ref[...], preferred_element_type=_F32) + b2_ref[...]
    h2_ref[...] = h2
    blk = jnp.concatenate(
        [jnp.sum(h2, axis=0, keepdims=True),
         jnp.sum(h2 * h2, axis=0, keepdims=True)], axis=0)

    @pl.when(pl.program_id(0) == 0)
    def _():
        st_ref[...] = blk

    @pl.when(pl.program_id(0) != 0)
    def _():
        st_ref[...] = st_ref[...] + blk


def _bn_apply(h2_ref, st_ref, g_ref, b_ref, res_ref, o_ref):
    inv_n = 1.0 / N
    mu = st_ref[0:1, :] * inv_n
    var = st_ref[1:2, :] * inv_n - mu * mu
    scale = lax.rsqrt(var + 1e-5) * g_ref[...]
    y = (h2_ref[...] - mu) * scale + b_ref[...]
    o_ref[...] = jnp.maximum(y, 0.0) + res_ref[...]


def _pool(b3_ref, h_ref, sum_ref, cnt_ref):
    seg = b3_ref[0, 0, :]
    onehot = (seg[:, None]
              == lax.broadcasted_iota(jnp.int32, (1, G), 1)).astype(_F32)
    sums = lax.dot_general(onehot, h_ref[...], (((0,), (0,)), ((), ())),
                           preferred_element_type=_F32)
    ones = jnp.ones((_RB, 1), _F32)
    cnts = lax.dot_general(onehot, ones, (((0,), (0,)), ((), ())),
                           preferred_element_type=_F32)

    @pl.when(pl.program_id(0) == 0)
    def _():
        sum_ref[...] = sums
        cnt_ref[...] = cnts

    @pl.when(pl.program_id(0) != 0)
    def _():
        sum_ref[...] = sum_ref[...] + sums
        cnt_ref[...] = cnt_ref[...] + cnts


def _head(sum_ref, cnt_ref, w1_ref, b1_ref, w2_ref, b2_ref,
          lo_ref, pr_ref, pd_ref):
    g = sum_ref[...] / jnp.maximum(cnt_ref[...], 1.0)
    gh = jnp.maximum(
        jnp.dot(g, w1_ref[...], preferred_element_type=_F32) + b1_ref[...], 0.0)
    logits = jnp.dot(gh, w2_ref[...], preferred_element_type=_F32) + b2_ref[...]
    probs = 1.0 / (1.0 + jnp.exp(-logits))
    preds = (probs > 0.5).astype(_F32)
    lo_ref[...] = logits
    pr_ref[...] = probs
    pd_ref[...] = preds


def _row_spec(nb):
    return pl.BlockSpec((nb, H), lambda i: (i, 0))


def kernel(x, edge_index, batch, edge_attr, Wn, bn_, We, be, eps, W1, b1,
           W2, b2, gamma, beta, Wc1, bc1, Wc2, bc2):
    f32 = _F32
    src = edge_index[0].reshape(NW, EPT)
    dst = edge_index[1].reshape(NW, EPT)
    pad = EPT_PAD - EPT
    srcp = jnp.pad(src, ((0, 0), (0, pad))).reshape(E_PAD)
    dstp = jnp.pad(dst, ((0, 0), (0, pad)),
                   constant_values=N).reshape(E_PAD)
    eap = jnp.pad(edge_attr.reshape(NW, EPT, DE),
                  ((0, 0), (0, pad), (0, 0))).reshape(E_PAD, DE)
    zblk = jnp.zeros((CH, H), f32)
    batch3 = batch.reshape(_NBLK, 1, _RB)
    wspec = pl.BlockSpec((H, H), lambda i: (0, 0))
    bspec = pl.BlockSpec((1, H), lambda i: (0, 0))

    h = pl.pallas_call(
        _proj_node,
        grid=(_NBLK,),
        in_specs=[_row_spec(_RB), wspec, bspec],
        out_specs=_row_spec(_RB),
        out_shape=jax.ShapeDtypeStruct((N, H), f32),
    )(x, Wn, bn_.reshape(1, H))

    ea = pl.pallas_call(
        _proj_edge,
        grid=(E_PAD // _EB,),
        in_specs=[pl.BlockSpec((_EB, DE), lambda i: (i, 0)),
                  pl.BlockSpec((DE, H), lambda i: (0, 0)), bspec],
        out_specs=_row_spec(_EB),
        out_shape=jax.ShapeDtypeStruct((E_PAD, H), f32),
    )(eap, We, be.reshape(1, H))

    scales = (1.0 + eps).reshape(L, 1, 1)
    b1r = b1.reshape(L, 1, H)
    b2r = b2.reshape(L, 1, H)
    gammar = gamma.reshape(L, 1, H)
    betar = beta.reshape(L, 1, H)

    for l in range(L):
        aggs = _make_message_pass()(h, ea, srcp, dstp, zblk)
        h2, st = pl.pallas_call(
            _mlp_stats,
            grid=(_NBLK,),
            in_specs=[
                pl.BlockSpec(memory_space=pltpu.SMEM),
                _row_spec(_RB),
                pl.BlockSpec((1, _RB, H), lambda i: (0, i, 0)),
                pl.BlockSpec((1, _RB, H), lambda i: (1, i, 0)),
                wspec, bspec, wspec, bspec,
            ],
            out_specs=[_row_spec(_RB), pl.BlockSpec((2, H), lambda i: (0, 0))],
            out_shape=[jax.ShapeDtypeStruct((N, H), f32),
                       jax.ShapeDtypeStruct((2, H), f32)],
        )(scales[l], h, aggs, aggs, W1[l], b1r[l], W2[l], b2r[l])

        h = pl.pallas_call(
            _bn_apply,
            grid=(_NBLK,),
            in_specs=[_row_spec(_RB),
                      pl.BlockSpec((2, H), lambda i: (0, 0)),
                      bspec, bspec, _row_spec(_RB)],
            out_specs=_row_spec(_RB),
            out_shape=jax.ShapeDtypeStruct((N, H), f32),
        )(h2, st, gammar[l], betar[l], h)

    sums, cnts = pl.pallas_call(
        _pool,
        grid=(_NBLK,),
        in_specs=[pl.BlockSpec((1, 1, _RB), lambda i: (i, 0, 0)),
                  _row_spec(_RB)],
        out_specs=[pl.BlockSpec((G, H), lambda i: (0, 0)),
                   pl.BlockSpec((G, 1), lambda i: (0, 0))],
        out_shape=[jax.ShapeDtypeStruct((G, H), f32),
                   jax.ShapeDtypeStruct((G, 1), f32)],
    )(batch3, h)

    logits, probs, preds = pl.pallas_call(
        _head,
        in_specs=[pl.BlockSpec((G, H), lambda: (0, 0)),
                  pl.BlockSpec((G, 1), lambda: (0, 0)),
                  pl.BlockSpec((H, H), lambda: (0, 0)),
                  pl.BlockSpec((1, H), lambda: (0, 0)),
                  pl.BlockSpec((H, C), lambda: (0, 0)),
                  pl.BlockSpec((1, C), lambda: (0, 0))],
        out_specs=[pl.BlockSpec((G, C), lambda: (0, 0))] * 3,
        out_shape=[jax.ShapeDtypeStruct((G, C), f32)] * 3,
    )(sums, cnts, Wc1, bc1.reshape(1, H), Wc2, bc2.reshape(1, C))

    return (logits, probs, preds, preds)


# speedup vs baseline: 4.3972x; 1.2270x over previous
import functools

import jax
import jax.numpy as jnp
from jax import lax
from jax.experimental import pallas as pl
from jax.experimental.pallas import tpu as pltpu
from jax.experimental.pallas import tpu_sc as plsc

N = 10000
E = 320000
D = 128
DE = 16
H = 128
L = 5
G = 128
C = 10

NC = 2
NS = 16
NW = NC * NS
CH = 80
EPT = E // NW
NCHUNK = 2 * (-(-EPT // (2 * CH)))
EPT_PAD = NCHUNK * CH
E_PAD = NW * EPT_PAD
AGG_ROWS = 10240
ZROWS = AGG_ROWS // NS

_F32 = jnp.float32


@functools.lru_cache(maxsize=None)
def _make_message_pass():
    mesh = plsc.VectorSubcoreMesh(
        core_axis_name="c", subcore_axis_name="s",
        num_cores=NC, num_subcores=NS)

    @functools.partial(
        pl.kernel,
        out_type=jax.ShapeDtypeStruct((NC, AGG_ROWS, H), _F32),
        mesh=mesh,
        scratch_types=[
            pltpu.VMEM_SHARED((AGG_ROWS, H), _F32),
            pltpu.VMEM((CH, H), _F32),
            pltpu.VMEM((CH, H), _F32),
            pltpu.VMEM((CH, H), _F32),
            pltpu.VMEM((CH, H), _F32),
            pltpu.VMEM((CH,), jnp.int32),
            pltpu.VMEM((CH,), jnp.int32),
            pltpu.VMEM((CH,), jnp.int32),
            pltpu.VMEM((CH,), jnp.int32),
            pltpu.VMEM((CH,), jnp.int32),
            pltpu.VMEM((CH,), jnp.int32),
        ] + [pltpu.SemaphoreType.DMA] * 12,
    )
    def message_pass(h_hbm, ea_hbm, src_hbm, dst_hbm, z_hbm, out_hbm,
                     agg, hb0, hb1, eb0, eb1, sv0, sv1, dv0, dv1, dv2, dv3,
                     gs0, gs1, es0, es1, ss0, ss1,
                     ds0, ds1, ds2, ds3, cs0, cs1):
        c = lax.axis_index("c")
        s = lax.axis_index("s")
        wid = c * NS + s

        pltpu.sync_copy(z_hbm, hb0)
        for k in range(ZROWS // CH):
            pltpu.sync_copy(hb0, agg.at[pl.ds(s * ZROWS + k * CH, CH)])
        plsc.subcore_barrier()

        base = wid * EPT_PAD
        hbufs = (hb0, hb1)
        ebufs = (eb0, eb1)
        srcv = (sv0, sv1)
        dstv = (dv0, dv1, dv2, dv3)
        gsems = (gs0, gs1)
        esems = (es0, es1)
        ssems = (ss0, ss1)
        dsems = (ds0, ds1, ds2, ds3)
        csems = (cs0, cs1)

        def sidx_copy(j, sl):
            return pltpu.make_async_copy(
                src_hbm.at[pl.ds(base + j * CH, CH)], srcv[sl], ssems[sl])

        def didx_copy(j, d4):
            return pltpu.make_async_copy(
                dst_hbm.at[pl.ds(base + j * CH, CH)], dstv[d4], dsems[d4])

        def gather_copy(sl):
            return pltpu.make_async_copy(
                h_hbm.at[srcv[sl]], hbufs[sl], gsems[sl])

        def ea_copy(j, sl):
            return pltpu.make_async_copy(
                ea_hbm.at[pl.ds(base + j * CH, CH)], ebufs[sl], esems[sl])

        def scat_copy(sl, d4):
            return pltpu.make_async_copy(
                ebufs[sl], agg.at[dstv[d4]], csems[sl])

        def compute(sl):
            hb = hbufs[sl]
            eb = ebufs[sl]

            def rows(r2, carry):
                r = r2 * 2
                for dr in range(2):
                    for q in range(H // 16):
                        sl_ = pl.ds(q * 16, 16)
                        v = hb[r + dr, sl_] + eb[r + dr, sl_]
                        eb[r + dr, sl_] = jnp.maximum(v, 0.0)
                return carry

            lax.fori_loop(0, CH // 2, rows, 0)

        def step(j, sl, d4, has_prev, has_next, has_next2):
            nsl = 1 - sl
            if has_next:
                sidx_copy(j + 1, nsl).wait()
                didx_copy(j + 1, (d4 + 1) % 4).wait()
                gather_copy(nsl).start()
            if has_next:
                ea_copy(j + 1, nsl).start()
            gather_copy(sl).wait()
            ea_copy(j, sl).wait()
            if has_next2:
                sidx_copy(j + 2, sl).start()
                didx_copy(j + 2, (d4 + 2) % 4).start()
            compute(sl)
            scat_copy(sl, d4).start(add=True)
            scat_copy(sl, d4).wait()

        sidx_copy(0, 0).start()
        didx_copy(0, 0).start()
        sidx_copy(1, 1).start()
        didx_copy(1, 1).start()
        sidx_copy(0, 0).wait()
        didx_copy(0, 0).wait()
        gather_copy(0).start()
        ea_copy(0, 0).start()
        step(0, 0, 0, False, True, True)
        step(1, 1, 1, True, True, True)

        def quad(i, carry):
            j0 = 4 * i + 2
            step(j0, 0, 2, True, True, True)
            step(j0 + 1, 1, 3, True, True, True)
            step(j0 + 2, 0, 0, True, True, True)
            step(j0 + 3, 1, 1, True, True, True)
            return carry

        assert NCHUNK % 4 == 2
        lax.fori_loop(0, (NCHUNK - 6) // 4, quad, 0)
        step(NCHUNK - 4, 0, 2, True, True, True)
        step(NCHUNK - 3, 1, 3, True, True, True)
        step(NCHUNK - 2, 0, 0, True, True, False)
        step(NCHUNK - 1, 1, 1, True, False, False)

        plsc.subcore_barrier()
        for k in range(ZROWS // CH):
            off = s * ZROWS + k * CH
            pltpu.sync_copy(agg.at[pl.ds(off, CH)], hb0)
            pltpu.sync_copy(hb0, out_hbm.at[c, pl.ds(off, CH)])

    return message_pass


_RB = 1000
_NBLK = N // _RB
_EB = 2520


def _proj_node(x_ref, w_ref, b_ref, o_ref):
    o_ref[...] = (
        jnp.dot(x_ref[...], w_ref[...], preferred_element_type=_F32)
        + b_ref[...])


def _proj_edge(a_ref, w_ref, b_ref, o_ref):
    o_ref[...] = (
        jnp.dot(a_ref[...], w_ref[...], preferred_element_type=_F32)
        + b_ref[...])


def _mlp_stats(scl_ref, h_ref, a0_ref, a1_ref, w1_ref, b1_ref, w2_ref, b2_ref,
               h2_ref, st_ref):
    z = scl_ref[0, 0] * h_ref[...] + a0_ref[0] + a1_ref[0]
    t = jnp.maximum(
        jnp.dot(z, w1_ref[...], preferred_element_type=_F32) + b1_ref[...], 0.0)
    h2 = jnp.dot(t, w2_ref[...], preferred_element_type=_F32) + b2_ref[...]
    h2_ref[...] = h2
    blk = jnp.concatenate(
        [jnp.sum(h2, axis=0, keepdims=True),
         jnp.sum(h2 * h2, axis=0, keepdims=True)], axis=0)

    @pl.when(pl.program_id(0) == 0)
    def _():
        st_ref[...] = blk

    @pl.when(pl.program_id(0) != 0)
    def _():
        st_ref[...] = st_ref[...] + blk


def _bn_apply(h2_ref, st_ref, g_ref, b_ref, res_ref, o_ref):
    inv_n = 1.0 / N
    mu = st_ref[0:1, :] * inv_n
    var = st_ref[1:2, :] * inv_n - mu * mu
    scale = lax.rsqrt(var + 1e-5) * g_ref[...]
    y = (h2_ref[...] - mu) * scale + b_ref[...]
    o_ref[...] = jnp.maximum(y, 0.0) + res_ref[...]


def _pool(b3_ref, h_ref, sum_ref, cnt_ref):
    seg = b3_ref[0, 0, :]
    onehot = (seg[:, None]
              == lax.broadcasted_iota(jnp.int32, (1, G), 1)).astype(_F32)
    sums = lax.dot_general(onehot, h_ref[...], (((0,), (0,)), ((), ())),
                           preferred_element_type=_F32)
    ones = jnp.ones((_RB, 1), _F32)
    cnts = lax.dot_general(onehot, ones, (((0,), (0,)), ((), ())),
                           preferred_element_type=_F32)

    @pl.when(pl.program_id(0) == 0)
    def _():
        sum_ref[...] = sums
        cnt_ref[...] = cnts

    @pl.when(pl.program_id(0) != 0)
    def _():
        sum_ref[...] = sum_ref[...] + sums
        cnt_ref[...] = cnt_ref[...] + cnts


def _head(sum_ref, cnt_ref, w1_ref, b1_ref, w2_ref, b2_ref,
          lo_ref, pr_ref, pd_ref):
    g = sum_ref[...] / jnp.maximum(cnt_ref[...], 1.0)
    gh = jnp.maximum(
        jnp.dot(g, w1_ref[...], preferred_element_type=_F32) + b1_ref[...], 0.0)
    logits = jnp.dot(gh, w2_ref[...], preferred_element_type=_F32) + b2_ref[...]
    probs = 1.0 / (1.0 + jnp.exp(-logits))
    preds = (probs > 0.5).astype(_F32)
    lo_ref[...] = logits
    pr_ref[...] = probs
    pd_ref[...] = preds


def _row_spec(nb):
    return pl.BlockSpec((nb, H), lambda i: (i, 0))


def kernel(x, edge_index, batch, edge_attr, Wn, bn_, We, be, eps, W1, b1,
           W2, b2, gamma, beta, Wc1, bc1, Wc2, bc2):
    f32 = _F32
    src = edge_index[0].reshape(NW, EPT)
    dst = edge_index[1].reshape(NW, EPT)
    pad = EPT_PAD - EPT
    srcp = jnp.pad(src, ((0, 0), (0, pad))).reshape(E_PAD)
    dstp = jnp.pad(dst, ((0, 0), (0, pad)),
                   constant_values=N).reshape(E_PAD)
    eap = jnp.pad(edge_attr.reshape(NW, EPT, DE),
                  ((0, 0), (0, pad), (0, 0))).reshape(E_PAD, DE)
    zblk = jnp.zeros((CH, H), f32)
    batch3 = batch.reshape(_NBLK, 1, _RB)
    wspec = pl.BlockSpec((H, H), lambda i: (0, 0))
    bspec = pl.BlockSpec((1, H), lambda i: (0, 0))

    h = pl.pallas_call(
        _proj_node,
        grid=(_NBLK,),
        in_specs=[_row_spec(_RB), wspec, bspec],
        out_specs=_row_spec(_RB),
        out_shape=jax.ShapeDtypeStruct((N, H), f32),
    )(x, Wn, bn_.reshape(1, H))

    ea = pl.pallas_call(
        _proj_edge,
        grid=(E_PAD // _EB,),
        in_specs=[pl.BlockSpec((_EB, DE), lambda i: (i, 0)),
                  pl.BlockSpec((DE, H), lambda i: (0, 0)), bspec],
        out_specs=_row_spec(_EB),
        out_shape=jax.ShapeDtypeStruct((E_PAD, H), f32),
    )(eap, We, be.reshape(1, H))

    scales = (1.0 + eps).reshape(L, 1, 1)
    b1r = b1.reshape(L, 1, H)
    b2r = b2.reshape(L, 1, H)
    gammar = gamma.reshape(L, 1, H)
    betar = beta.reshape(L, 1, H)

    for l in range(L):
        aggs = _make_message_pass()(h, ea, srcp, dstp, zblk)
        h2, st = pl.pallas_call(
            _mlp_stats,
            grid=(_NBLK,),
            in_specs=[
                pl.BlockSpec(memory_space=pltpu.SMEM),
                _row_spec(_RB),
                pl.BlockSpec((1, _RB, H), lambda i: (0, i, 0)),
                pl.BlockSpec((1, _RB, H), lambda i: (1, i, 0)),
                wspec, bspec, wspec, bspec,
            ],
            out_specs=[_row_spec(_RB), pl.BlockSpec((2, H), lambda i: (0, 0))],
            out_shape=[jax.ShapeDtypeStruct((N, H), f32),
                       jax.ShapeDtypeStruct((2, H), f32)],
        )(scales[l], h, aggs, aggs, W1[l], b1r[l], W2[l], b2r[l])

        h = pl.pallas_call(
            _bn_apply,
            grid=(_NBLK,),
            in_specs=[_row_spec(_RB),
                      pl.BlockSpec((2, H), lambda i: (0, 0)),
                      bspec, bspec, _row_spec(_RB)],
            out_specs=_row_spec(_RB),
            out_shape=jax.ShapeDtypeStruct((N, H), f32),
        )(h2, st, gammar[l], betar[l], h)

    sums, cnts = pl.pallas_call(
        _pool,
        grid=(_NBLK,),
        in_specs=[pl.BlockSpec((1, 1, _RB), lambda i: (i, 0, 0)),
                  _row_spec(_RB)],
        out_specs=[pl.BlockSpec((G, H), lambda i: (0, 0)),
                   pl.BlockSpec((G, 1), lambda i: (0, 0))],
        out_shape=[jax.ShapeDtypeStruct((G, H), f32),
                   jax.ShapeDtypeStruct((G, 1), f32)],
    )(batch3, h)

    logits, probs, preds = pl.pallas_call(
        _head,
        in_specs=[pl.BlockSpec((G, H), lambda: (0, 0)),
                  pl.BlockSpec((G, 1), lambda: (0, 0)),
                  pl.BlockSpec((H, H), lambda: (0, 0)),
                  pl.BlockSpec((1, H), lambda: (0, 0)),
                  pl.BlockSpec((H, C), lambda: (0, 0)),
                  pl.BlockSpec((1, C), lambda: (0, 0))],
        out_specs=[pl.BlockSpec((G, C), lambda: (0, 0))] * 3,
        out_shape=[jax.ShapeDtypeStruct((G, C), f32)] * 3,
    )(sums, cnts, Wc1, bc1.reshape(1, H), Wc2, bc2.reshape(1, C))

    return (logits, probs, preds, preds)
